# baseline (speedup 1.0000x reference)
import functools

import numpy as np
import jax
import jax.numpy as jnp
from jax import lax
from jax.experimental import pallas as pl
from jax.experimental.pallas import tpu as pltpu
from jax.experimental.pallas import tpu_sc as plsc

_B, _N, _D = 4, 1024, 256
_HEADS, _DH, _P = 8, 32, 4
_H, _W = 200, 200
_HW = _H * _W
_K = 10
_S = _B * 4 * _N * 32


def _consts():
    t = np.linspace(0.0, 1.0, _K)
    coeff = np.stack([(1 - t) ** 3, 3 * (1 - t) ** 2 * t,
                      3 * (1 - t) * t ** 2, t ** 3], -1)
    m64 = np.zeros((8, 64), np.float32)
    for c in range(4):
        for k in range(_K):
            m64[2 * c, k] = coeff[k, c]
            m64[2 * c + 1, 32 + k] = coeff[k, c]
    operm = np.zeros(64, np.int32)
    for h in range(_HEADS):
        for p in range(_P):
            for comp in range(2):
                operm[comp * 32 + p * 8 + h] = h * 8 + p * 2 + comp
    aperm = np.zeros(32, np.int32)
    for h in range(_HEADS):
        for p in range(_P):
            aperm[p * 8 + h] = h * 4 + p
    e2 = np.zeros((32, 4096), np.float32)
    for s in range(32):
        e2[s, s * 128:(s + 1) * 128] = 1.0
    rows, srcs = [], []
    for s in range(32):
        h = s % 8
        for j in range(32):
            rows.append(s * 128 + (h % 4) * 32 + j)
            srcs.append(h * 32 + j)
    return m64, operm, aperm, e2, np.array(rows), np.array(srcs)


_M64_np, _OPERM_np, _APERM_np, _E2_np, _SELROWS_np, _SELSRCS_np = _consts()


_ABLK = 8000


def _value_body(x_ref, w_ref, b_ref, o_ref):
    o_ref[0] = (jnp.dot(x_ref[0], w_ref[...],
                        preferred_element_type=jnp.float32) + b_ref[...])


def _value_proj(value_in, wv_t, bv):
    return pl.pallas_call(
        _value_body,
        grid=(_B, _HW // _ABLK),
        in_specs=[
            pl.BlockSpec((1, _ABLK, _D), lambda b, i: (b, i, 0)),
            pl.BlockSpec((_D, _D), lambda b, i: (0, 0)),
            pl.BlockSpec((1, _D), lambda b, i: (0, 0)),
        ],
        out_specs=pl.BlockSpec((1, _ABLK, _D), lambda b, i: (b, i, 0)),
        out_shape=jax.ShapeDtypeStruct((_B, _HW, _D), jnp.float32),
        compiler_params=pltpu.CompilerParams(
            dimension_semantics=("parallel", "parallel"),
            allow_input_fusion=[True, False, False]),
    )(value_in, wv_t, bv)


def _query_body(scal_ref, qe_ref, ctrl_ref, wq_ref, bq_ref, woff_ref, boff_ref,
                wattn_ref, battn_ref, m64_ref, q_ref, idx_ref, wts_ref):
    b = pl.program_id(0)
    q = (jnp.dot(qe_ref[0], wq_ref[...],
                 preferred_element_type=jnp.float32) + bq_ref[...])
    q_ref[0] = q
    off = (jnp.dot(q, woff_ref[...],
                   preferred_element_type=jnp.float32) + boff_ref[...])
    al = (jnp.dot(q, wattn_ref[...],
                  preferred_element_type=jnp.float32) + battn_ref[...])
    parts = [al[:, 8 * p:8 * (p + 1)] for p in range(4)]
    m = jnp.maximum(jnp.maximum(parts[0], parts[1]),
                    jnp.maximum(parts[2], parts[3]))
    es = [jnp.exp(p_ - m) for p_ in parts]
    denom = es[0] + es[1] + es[2] + es[3]
    aw = jnp.concatenate([e / denom for e in es], axis=1)

    dense = jnp.dot(ctrl_ref[0], m64_ref[...],
                    preferred_element_type=jnp.float32)
    lane = lax.broadcasted_iota(jnp.int32, (_N, 32), 1)
    kmask = lane < _K
    xn = (dense[:, 0:32] - scal_ref[0]) * scal_ref[1]
    yn = (dense[:, 32:64] - scal_ref[2]) * scal_ref[3]
    xn = jnp.where(kmask, jnp.clip(xn, 0.01, 0.99), 0.0)
    yn = jnp.where(kmask, jnp.clip(yn, 0.01, 0.99), 0.0)
    rcx = jnp.sum(xn, axis=1, keepdims=True) * (1.0 / _K)
    rcy = jnp.sum(yn, axis=1, keepdims=True) * (1.0 / _K)

    px = rcx * _W + off[:, 0:32] - 0.5
    py = rcy * _H + off[:, 32:64] - 0.5
    x0 = jnp.floor(px)
    y0 = jnp.floor(py)
    wx1 = px - x0
    wx0 = 1.0 - wx1
    wy1 = py - y0
    wy0 = 1.0 - wy1
    gvec = (lane % 8) // 4
    corners = ((x0, y0, wx0 * wy0), (x0 + 1.0, y0, wx1 * wy0),
               (x0, y0 + 1.0, wx0 * wy1), (x0 + 1.0, y0 + 1.0, wx1 * wy1))
    for c, (cx, cy, wgt) in enumerate(corners):
        valid = ((cx >= 0.0) & (cx <= _W - 1.0)
                 & (cy >= 0.0) & (cy <= _H - 1.0))
        ix = jnp.clip(cx, 0.0, _W - 1.0).astype(jnp.int32)
        iy = jnp.clip(cy, 0.0, _H - 1.0).astype(jnp.int32)
        idx_ref[0, c] = (b * _HW + iy * _W + ix) * 2 + gvec
        wts_ref[0, c] = jnp.where(valid, wgt, 0.0) * aw


def _query_side(scal, qe, ctrl8, wq_t, bq, woff_t, boff, wattn_t, battn, m64):
    full = lambda shape: pl.BlockSpec(shape, lambda b: (0,) * len(shape))
    return pl.pallas_call(
        _query_body,
        grid=(_B,),
        in_specs=[
            pl.BlockSpec(memory_space=pltpu.SMEM),
            pl.BlockSpec((1, _N, _D), lambda b: (b, 0, 0)),
            pl.BlockSpec((1, _N, 8), lambda b: (b, 0, 0)),
            full((_D, _D)), full((1, _D)),
            full((_D, 64)), full((1, 64)),
            full((_D, 32)), full((1, 32)),
            full((8, 64)),
        ],
        out_specs=[
            pl.BlockSpec((1, _N, _D), lambda b: (b, 0, 0)),
            pl.BlockSpec((1, 4, _N, 32), lambda b: (b, 0, 0, 0)),
            pl.BlockSpec((1, 4, _N, 32), lambda b: (b, 0, 0, 0)),
        ],
        out_shape=[
            jax.ShapeDtypeStruct((_B, _N, _D), jnp.float32),
            jax.ShapeDtypeStruct((_B, 4, _N, 32), jnp.int32),
            jax.ShapeDtypeStruct((_B, 4, _N, 32), jnp.float32),
        ],
        compiler_params=pltpu.CompilerParams(
            dimension_semantics=("parallel",)),
    )(scal, qe, ctrl8, wq_t, bq, woff_t, boff, wattn_t, battn, m64)


_NW = 32
_PER_W = _S // _NW
_CH = 256


def _sc_gather_body(table_ref, idx_ref, out_ref, idx_v, rows_v,
                    semg0, semg1, semw0, semw1):
    wid = lax.axis_index("s") * 2 + lax.axis_index("c")
    base = pl.multiple_of(wid * _PER_W, _CH)
    pltpu.sync_copy(idx_ref.at[pl.ds(base, _PER_W)], idx_v)
    semg = (semg0, semg1)
    semw = (semw0, semw1)
    nch = _PER_W // _CH
    pend_g = [None, None]
    pend_w = [None, None]
    offs = [None, None]
    for j in range(nch):
        cur = j % 2
        prv = 1 - cur
        off = pl.multiple_of(base + j * _CH, _CH)
        if pend_w[cur] is not None:
            pend_w[cur].wait()
            pend_w[cur] = None
        pend_g[cur] = pltpu.async_copy(
            table_ref.at[idx_v.at[pl.ds(j * _CH, _CH)]],
            rows_v.at[cur], semg[cur])
        offs[cur] = off
        if pend_g[prv] is not None:
            pend_g[prv].wait()
            pend_g[prv] = None
            pend_w[prv] = pltpu.async_copy(
                rows_v.at[prv], out_ref.at[pl.ds(offs[prv], _CH)], semw[prv])
    last = (nch - 1) % 2
    pend_g[last].wait()
    pend_w[last] = pltpu.async_copy(
        rows_v.at[last], out_ref.at[pl.ds(offs[last], _CH)], semw[last])
    for bufi in range(2):
        if pend_w[bufi] is not None:
            pend_w[bufi].wait()


_sc_gather = functools.partial(
    pl.kernel,
    out_type=jax.ShapeDtypeStruct((_S, 128), jnp.float32),
    mesh=plsc.VectorSubcoreMesh(core_axis_name="c", subcore_axis_name="s"),
    scratch_types=[
        pltpu.VMEM((_PER_W,), jnp.int32),
        pltpu.VMEM((2, _CH, 128), jnp.float32),
        pltpu.SemaphoreType.DMA,
        pltpu.SemaphoreType.DMA,
        pltpu.SemaphoreType.DMA,
        pltpu.SemaphoreType.DMA,
    ],
)(_sc_gather_body)


_NB = 256


def _combine_body(g_ref, w_ref, e2_ref, q_ref, selwm_ref, bm_ref, wo_ref,
                  bo_ref, o_ref):
    u = g_ref[0, 0] * jnp.dot(w_ref[0, 0], e2_ref[...],
                              preferred_element_type=jnp.float32)
    for c in range(1, 4):
        u = u + g_ref[0, c] * jnp.dot(w_ref[0, c], e2_ref[...],
                                      preferred_element_type=jnp.float32)
    y = (jnp.dot(u, selwm_ref[...], preferred_element_type=jnp.float32)
         + bm_ref[...] + q_ref[0])
    o_ref[0] = (jnp.dot(y, wo_ref[...], preferred_element_type=jnp.float32)
                + bo_ref[...])


def _combine(g4, wts, e2, q, selwm, bm, wo_t, bo):
    full = lambda shape: pl.BlockSpec(shape, lambda b, i: (0,) * len(shape))
    return pl.pallas_call(
        _combine_body,
        grid=(_B, _N // _NB),
        in_specs=[
            pl.BlockSpec((1, 4, _NB, 4096), lambda b, i: (b, 0, i, 0)),
            pl.BlockSpec((1, 4, _NB, 32), lambda b, i: (b, 0, i, 0)),
            full((32, 4096)),
            pl.BlockSpec((1, _NB, _D), lambda b, i: (b, i, 0)),
            full((4096, _D)), full((1, _D)),
            full((_D, _D)), full((1, _D)),
        ],
        out_specs=pl.BlockSpec((1, _NB, _D), lambda b, i: (b, i, 0)),
        out_shape=jax.ShapeDtypeStruct((_B, _N, _D), jnp.float32),
        compiler_params=pltpu.CompilerParams(
            dimension_semantics=("parallel", "parallel")),
    )(g4, wts, e2, q, selwm, bm, wo_t, bo)


def kernel(query_embed, ctrl_points, bev_features, spatial_shapes, pc_range,
           W_query, b_query, W_value, b_value, W_off, b_off,
           W_attn, b_attn, W_msda_out, b_msda_out, W_out, b_out):
    del spatial_shapes
    m64 = jnp.asarray(_M64_np)
    e2 = jnp.asarray(_E2_np)
    operm = jnp.asarray(_OPERM_np)
    aperm = jnp.asarray(_APERM_np)
    selwm = (jnp.zeros((4096, _D), jnp.float32)
             .at[_SELROWS_np].set(W_msda_out.T[_SELSRCS_np]))

    value_in = bev_features.reshape(_B, _D, _HW).transpose(0, 2, 1)
    value = _value_proj(value_in, W_value.T, b_value[None])
    table = value.reshape(_B * _HW * 2, 128)

    scal = jnp.stack([pc_range[0], 1.0 / (pc_range[3] - pc_range[0]),
                      pc_range[1], 1.0 / (pc_range[4] - pc_range[1])])
    ctrl8 = ctrl_points.reshape(_B, _N, 8)
    q, idx, wts = _query_side(
        scal, query_embed, ctrl8,
        W_query.T, b_query[None],
        W_off[operm].T, b_off[operm][None],
        W_attn[aperm].T, b_attn[aperm][None],
        m64)

    g = _sc_gather(table, idx.reshape(_S))
    g4 = g.reshape(_B, 4, _N, 32 * 128)

    return _combine(g4, wts, e2, q, selwm, b_msda_out[None], W_out.T, b_out[None])


# speedup vs baseline: 1.0013x; 1.0013x over previous
import functools

import numpy as np
import jax
import jax.numpy as jnp
from jax import lax
from jax.experimental import pallas as pl
from jax.experimental.pallas import tpu as pltpu
from jax.experimental.pallas import tpu_sc as plsc

_B, _N, _D = 4, 1024, 256
_HEADS, _DH, _P = 8, 32, 4
_H, _W = 200, 200
_HW = _H * _W
_K = 10
_S = _B * 4 * _N * 32


def _consts():
    t = np.linspace(0.0, 1.0, _K)
    coeff = np.stack([(1 - t) ** 3, 3 * (1 - t) ** 2 * t,
                      3 * (1 - t) * t ** 2, t ** 3], -1)
    m64 = np.zeros((8, 64), np.float32)
    for c in range(4):
        for k in range(_K):
            m64[2 * c, k] = coeff[k, c]
            m64[2 * c + 1, 32 + k] = coeff[k, c]
    operm = np.zeros(64, np.int32)
    for h in range(_HEADS):
        for p in range(_P):
            for comp in range(2):
                operm[comp * 32 + p * 8 + h] = h * 8 + p * 2 + comp
    aperm = np.zeros(32, np.int32)
    for h in range(_HEADS):
        for p in range(_P):
            aperm[p * 8 + h] = h * 4 + p
    e2 = np.zeros((32, 4096), np.float32)
    for s in range(32):
        e2[s, s * 128:(s + 1) * 128] = 1.0
    rows, srcs = [], []
    for s in range(32):
        h = s % 8
        for j in range(32):
            rows.append(s * 128 + (h % 4) * 32 + j)
            srcs.append(h * 32 + j)
    return m64, operm, aperm, e2, np.array(rows), np.array(srcs)


_M64_np, _OPERM_np, _APERM_np, _E2_np, _SELROWS_np, _SELSRCS_np = _consts()


_ABLK = 5000


def _value_body(x_ref, w_ref, b_ref, o_ref):
    o_ref[0] = (jnp.dot(x_ref[0], w_ref[...],
                        preferred_element_type=jnp.float32) + b_ref[...])


def _value_proj(value_in, wv_t, bv):
    return pl.pallas_call(
        _value_body,
        grid=(_B, _HW // _ABLK),
        in_specs=[
            pl.BlockSpec((1, _ABLK, _D), lambda b, i: (b, i, 0)),
            pl.BlockSpec((_D, _D), lambda b, i: (0, 0)),
            pl.BlockSpec((1, _D), lambda b, i: (0, 0)),
        ],
        out_specs=pl.BlockSpec((1, _ABLK, _D), lambda b, i: (b, i, 0)),
        out_shape=jax.ShapeDtypeStruct((_B, _HW, _D), jnp.float32),
        compiler_params=pltpu.CompilerParams(
            dimension_semantics=("parallel", "parallel")),
    )(value_in, wv_t, bv)


def _query_body(scal_ref, qe_ref, ctrl_ref, wq_ref, bq_ref, woff_ref, boff_ref,
                wattn_ref, battn_ref, m64_ref, q_ref, idx_ref, wts_ref):
    b = pl.program_id(0)
    q = (jnp.dot(qe_ref[0], wq_ref[...],
                 preferred_element_type=jnp.float32) + bq_ref[...])
    q_ref[0] = q
    off = (jnp.dot(q, woff_ref[...],
                   preferred_element_type=jnp.float32) + boff_ref[...])
    al = (jnp.dot(q, wattn_ref[...],
                  preferred_element_type=jnp.float32) + battn_ref[...])
    parts = [al[:, 8 * p:8 * (p + 1)] for p in range(4)]
    m = jnp.maximum(jnp.maximum(parts[0], parts[1]),
                    jnp.maximum(parts[2], parts[3]))
    es = [jnp.exp(p_ - m) for p_ in parts]
    denom = es[0] + es[1] + es[2] + es[3]
    aw = jnp.concatenate([e / denom for e in es], axis=1)

    dense = jnp.dot(ctrl_ref[0], m64_ref[...],
                    preferred_element_type=jnp.float32)
    lane = lax.broadcasted_iota(jnp.int32, (_N, 32), 1)
    kmask = lane < _K
    xn = (dense[:, 0:32] - scal_ref[0]) * scal_ref[1]
    yn = (dense[:, 32:64] - scal_ref[2]) * scal_ref[3]
    xn = jnp.where(kmask, jnp.clip(xn, 0.01, 0.99), 0.0)
    yn = jnp.where(kmask, jnp.clip(yn, 0.01, 0.99), 0.0)
    rcx = jnp.sum(xn, axis=1, keepdims=True) * (1.0 / _K)
    rcy = jnp.sum(yn, axis=1, keepdims=True) * (1.0 / _K)

    px = rcx * _W + off[:, 0:32] - 0.5
    py = rcy * _H + off[:, 32:64] - 0.5
    x0 = jnp.floor(px)
    y0 = jnp.floor(py)
    wx1 = px - x0
    wx0 = 1.0 - wx1
    wy1 = py - y0
    wy0 = 1.0 - wy1
    gvec = (lane % 8) // 4
    corners = ((x0, y0, wx0 * wy0), (x0 + 1.0, y0, wx1 * wy0),
               (x0, y0 + 1.0, wx0 * wy1), (x0 + 1.0, y0 + 1.0, wx1 * wy1))
    for c, (cx, cy, wgt) in enumerate(corners):
        valid = ((cx >= 0.0) & (cx <= _W - 1.0)
                 & (cy >= 0.0) & (cy <= _H - 1.0))
        ix = jnp.clip(cx, 0.0, _W - 1.0).astype(jnp.int32)
        iy = jnp.clip(cy, 0.0, _H - 1.0).astype(jnp.int32)
        idx_ref[0, c] = (b * _HW + iy * _W + ix) * 2 + gvec
        wts_ref[0, c] = jnp.where(valid, wgt, 0.0) * aw


def _query_side(scal, qe, ctrl8, wq_t, bq, woff_t, boff, wattn_t, battn, m64):
    full = lambda shape: pl.BlockSpec(shape, lambda b: (0,) * len(shape))
    return pl.pallas_call(
        _query_body,
        grid=(_B,),
        in_specs=[
            pl.BlockSpec(memory_space=pltpu.SMEM),
            pl.BlockSpec((1, _N, _D), lambda b: (b, 0, 0)),
            pl.BlockSpec((1, _N, 8), lambda b: (b, 0, 0)),
            full((_D, _D)), full((1, _D)),
            full((_D, 64)), full((1, 64)),
            full((_D, 32)), full((1, 32)),
            full((8, 64)),
        ],
        out_specs=[
            pl.BlockSpec((1, _N, _D), lambda b: (b, 0, 0)),
            pl.BlockSpec((1, 4, _N, 32), lambda b: (b, 0, 0, 0)),
            pl.BlockSpec((1, 4, _N, 32), lambda b: (b, 0, 0, 0)),
        ],
        out_shape=[
            jax.ShapeDtypeStruct((_B, _N, _D), jnp.float32),
            jax.ShapeDtypeStruct((_B, 4, _N, 32), jnp.int32),
            jax.ShapeDtypeStruct((_B, 4, _N, 32), jnp.float32),
        ],
        compiler_params=pltpu.CompilerParams(
            dimension_semantics=("parallel",)),
    )(scal, qe, ctrl8, wq_t, bq, woff_t, boff, wattn_t, battn, m64)


_NW = 32
_PER_W = _S // _NW
_CH = 256


def _sc_gather_body(table_ref, idx_ref, out_ref, idx_v, rows_v,
                    semg0, semg1, semw0, semw1):
    wid = lax.axis_index("s") * 2 + lax.axis_index("c")
    base = pl.multiple_of(wid * _PER_W, _CH)
    pltpu.sync_copy(idx_ref.at[pl.ds(base, _PER_W)], idx_v)
    semg = (semg0, semg1)
    semw = (semw0, semw1)
    nch = _PER_W // _CH
    pend_g = [None, None]
    pend_w = [None, None]
    offs = [None, None]
    for j in range(nch):
        cur = j % 2
        prv = 1 - cur
        off = pl.multiple_of(base + j * _CH, _CH)
        if pend_w[cur] is not None:
            pend_w[cur].wait()
            pend_w[cur] = None
        pend_g[cur] = pltpu.async_copy(
            table_ref.at[idx_v.at[pl.ds(j * _CH, _CH)]],
            rows_v.at[cur], semg[cur])
        offs[cur] = off
        if pend_g[prv] is not None:
            pend_g[prv].wait()
            pend_g[prv] = None
            pend_w[prv] = pltpu.async_copy(
                rows_v.at[prv], out_ref.at[pl.ds(offs[prv], _CH)], semw[prv])
    last = (nch - 1) % 2
    pend_g[last].wait()
    pend_w[last] = pltpu.async_copy(
        rows_v.at[last], out_ref.at[pl.ds(offs[last], _CH)], semw[last])
    for bufi in range(2):
        if pend_w[bufi] is not None:
            pend_w[bufi].wait()


_sc_gather = functools.partial(
    pl.kernel,
    out_type=jax.ShapeDtypeStruct((_S, 128), jnp.float32),
    mesh=plsc.VectorSubcoreMesh(core_axis_name="c", subcore_axis_name="s"),
    scratch_types=[
        pltpu.VMEM((_PER_W,), jnp.int32),
        pltpu.VMEM((2, _CH, 128), jnp.float32),
        pltpu.SemaphoreType.DMA,
        pltpu.SemaphoreType.DMA,
        pltpu.SemaphoreType.DMA,
        pltpu.SemaphoreType.DMA,
    ],
)(_sc_gather_body)


_NB = 256


def _combine_body(g_ref, w_ref, e2_ref, q_ref, selwm_ref, bm_ref, wo_ref,
                  bo_ref, o_ref):
    u = g_ref[0, 0] * jnp.dot(w_ref[0, 0], e2_ref[...],
                              preferred_element_type=jnp.float32)
    for c in range(1, 4):
        u = u + g_ref[0, c] * jnp.dot(w_ref[0, c], e2_ref[...],
                                      preferred_element_type=jnp.float32)
    y = (jnp.dot(u, selwm_ref[...], preferred_element_type=jnp.float32)
         + bm_ref[...] + q_ref[0])
    o_ref[0] = (jnp.dot(y, wo_ref[...], preferred_element_type=jnp.float32)
                + bo_ref[...])


def _combine(g4, wts, e2, q, selwm, bm, wo_t, bo):
    full = lambda shape: pl.BlockSpec(shape, lambda b, i: (0,) * len(shape))
    return pl.pallas_call(
        _combine_body,
        grid=(_B, _N // _NB),
        in_specs=[
            pl.BlockSpec((1, 4, _NB, 4096), lambda b, i: (b, 0, i, 0)),
            pl.BlockSpec((1, 4, _NB, 32), lambda b, i: (b, 0, i, 0)),
            full((32, 4096)),
            pl.BlockSpec((1, _NB, _D), lambda b, i: (b, i, 0)),
            full((4096, _D)), full((1, _D)),
            full((_D, _D)), full((1, _D)),
        ],
        out_specs=pl.BlockSpec((1, _NB, _D), lambda b, i: (b, i, 0)),
        out_shape=jax.ShapeDtypeStruct((_B, _N, _D), jnp.float32),
        compiler_params=pltpu.CompilerParams(
            dimension_semantics=("parallel", "parallel")),
    )(g4, wts, e2, q, selwm, bm, wo_t, bo)


def kernel(query_embed, ctrl_points, bev_features, spatial_shapes, pc_range,
           W_query, b_query, W_value, b_value, W_off, b_off,
           W_attn, b_attn, W_msda_out, b_msda_out, W_out, b_out):
    del spatial_shapes
    m64 = jnp.asarray(_M64_np)
    e2 = jnp.asarray(_E2_np)
    operm = jnp.asarray(_OPERM_np)
    aperm = jnp.asarray(_APERM_np)
    selwm = (jnp.zeros((4096, _D), jnp.float32)
             .at[_SELROWS_np].set(W_msda_out.T[_SELSRCS_np]))

    value_in = bev_features.reshape(_B, _D, _HW).transpose(0, 2, 1)
    value = _value_proj(value_in, W_value.T, b_value[None])
    table = value.reshape(_B * _HW * 2, 128)

    scal = jnp.stack([pc_range[0], 1.0 / (pc_range[3] - pc_range[0]),
                      pc_range[1], 1.0 / (pc_range[4] - pc_range[1])])
    ctrl8 = ctrl_points.reshape(_B, _N, 8)
    q, idx, wts = _query_side(
        scal, query_embed, ctrl8,
        W_query.T, b_query[None],
        W_off[operm].T, b_off[operm][None],
        W_attn[aperm].T, b_attn[aperm][None],
        m64)

    g = _sc_gather(table, idx.reshape(_S))
    g4 = g.reshape(_B, 4, _N, 32 * 128)

    return _combine(g4, wts, e2, q, selwm, b_msda_out[None], W_out.T, b_out[None])


# speedup vs baseline: 1.0221x; 1.0207x over previous
import functools

import numpy as np
import jax
import jax.numpy as jnp
from jax import lax
from jax.experimental import pallas as pl
from jax.experimental.pallas import tpu as pltpu
from jax.experimental.pallas import tpu_sc as plsc

_B, _N, _D = 4, 1024, 256
_HEADS, _DH, _P = 8, 32, 4
_H, _W = 200, 200
_HW = _H * _W
_K = 10
_S = _B * 4 * _N * 32


def _consts():
    t = np.linspace(0.0, 1.0, _K)
    coeff = np.stack([(1 - t) ** 3, 3 * (1 - t) ** 2 * t,
                      3 * (1 - t) * t ** 2, t ** 3], -1)
    m64 = np.zeros((8, 64), np.float32)
    for c in range(4):
        for k in range(_K):
            m64[2 * c, k] = coeff[k, c]
            m64[2 * c + 1, 32 + k] = coeff[k, c]
    operm = np.zeros(64, np.int32)
    for h in range(_HEADS):
        for p in range(_P):
            for comp in range(2):
                operm[comp * 32 + p * 8 + h] = h * 8 + p * 2 + comp
    aperm = np.zeros(32, np.int32)
    for h in range(_HEADS):
        for p in range(_P):
            aperm[p * 8 + h] = h * 4 + p
    e2 = np.zeros((32, 4096), np.float32)
    for s in range(32):
        e2[s, s * 128:(s + 1) * 128] = 1.0
    rows, srcs = [], []
    for s in range(32):
        h = s % 8
        for j in range(32):
            rows.append(s * 128 + (h % 4) * 32 + j)
            srcs.append(h * 32 + j)
    return m64, operm, aperm, e2, np.array(rows), np.array(srcs)


_M64_np, _OPERM_np, _APERM_np, _E2_np, _SELROWS_np, _SELSRCS_np = _consts()


_ABLK = 5000


def _value_body(x_ref, w_ref, b_ref, o_ref):
    o_ref[0] = (jnp.dot(x_ref[0], w_ref[...],
                        preferred_element_type=jnp.float32) + b_ref[...])


def _value_proj(value_in, wv_t, bv):
    return pl.pallas_call(
        _value_body,
        grid=(_B, _HW // _ABLK),
        in_specs=[
            pl.BlockSpec((1, _ABLK, _D), lambda b, i: (b, i, 0)),
            pl.BlockSpec((_D, _D), lambda b, i: (0, 0)),
            pl.BlockSpec((1, _D), lambda b, i: (0, 0)),
        ],
        out_specs=pl.BlockSpec((1, _ABLK, _D), lambda b, i: (b, i, 0)),
        out_shape=jax.ShapeDtypeStruct((_B, _HW, _D), jnp.float32),
        compiler_params=pltpu.CompilerParams(
            dimension_semantics=("parallel", "parallel")),
    )(value_in, wv_t, bv)


def _query_body(scal_ref, qe_ref, ctrl_ref, wq_ref, bq_ref, woff_ref, boff_ref,
                wattn_ref, battn_ref, m64_ref, q_ref, idx_ref, wts_ref):
    b = pl.program_id(0)
    q = (jnp.dot(qe_ref[0], wq_ref[...],
                 preferred_element_type=jnp.float32) + bq_ref[...])
    q_ref[0] = q
    off = (jnp.dot(q, woff_ref[...],
                   preferred_element_type=jnp.float32) + boff_ref[...])
    al = (jnp.dot(q, wattn_ref[...],
                  preferred_element_type=jnp.float32) + battn_ref[...])
    parts = [al[:, 8 * p:8 * (p + 1)] for p in range(4)]
    m = jnp.maximum(jnp.maximum(parts[0], parts[1]),
                    jnp.maximum(parts[2], parts[3]))
    es = [jnp.exp(p_ - m) for p_ in parts]
    denom = es[0] + es[1] + es[2] + es[3]
    aw = jnp.concatenate([e / denom for e in es], axis=1)

    dense = jnp.dot(ctrl_ref[0], m64_ref[...],
                    preferred_element_type=jnp.float32)
    lane = lax.broadcasted_iota(jnp.int32, (_N, 32), 1)
    kmask = lane < _K
    xn = (dense[:, 0:32] - scal_ref[0]) * scal_ref[1]
    yn = (dense[:, 32:64] - scal_ref[2]) * scal_ref[3]
    xn = jnp.where(kmask, jnp.clip(xn, 0.01, 0.99), 0.0)
    yn = jnp.where(kmask, jnp.clip(yn, 0.01, 0.99), 0.0)
    rcx = jnp.sum(xn, axis=1, keepdims=True) * (1.0 / _K)
    rcy = jnp.sum(yn, axis=1, keepdims=True) * (1.0 / _K)

    px = rcx * _W + off[:, 0:32] - 0.5
    py = rcy * _H + off[:, 32:64] - 0.5
    x0 = jnp.floor(px)
    y0 = jnp.floor(py)
    wx1 = px - x0
    wx0 = 1.0 - wx1
    wy1 = py - y0
    wy0 = 1.0 - wy1
    gvec = (lane % 8) // 4
    corners = ((x0, y0, wx0 * wy0), (x0 + 1.0, y0, wx1 * wy0),
               (x0, y0 + 1.0, wx0 * wy1), (x0 + 1.0, y0 + 1.0, wx1 * wy1))
    for c, (cx, cy, wgt) in enumerate(corners):
        valid = ((cx >= 0.0) & (cx <= _W - 1.0)
                 & (cy >= 0.0) & (cy <= _H - 1.0))
        ix = jnp.clip(cx, 0.0, _W - 1.0).astype(jnp.int32)
        iy = jnp.clip(cy, 0.0, _H - 1.0).astype(jnp.int32)
        idx_ref[0, c] = (b * _HW + iy * _W + ix) * 2 + gvec
        wts_ref[0, c] = jnp.where(valid, wgt, 0.0) * aw


def _query_side(scal, qe, ctrl8, wq_t, bq, woff_t, boff, wattn_t, battn, m64):
    full = lambda shape: pl.BlockSpec(shape, lambda b: (0,) * len(shape))
    return pl.pallas_call(
        _query_body,
        grid=(_B,),
        in_specs=[
            pl.BlockSpec(memory_space=pltpu.SMEM),
            pl.BlockSpec((1, _N, _D), lambda b: (b, 0, 0)),
            pl.BlockSpec((1, _N, 8), lambda b: (b, 0, 0)),
            full((_D, _D)), full((1, _D)),
            full((_D, 64)), full((1, 64)),
            full((_D, 32)), full((1, 32)),
            full((8, 64)),
        ],
        out_specs=[
            pl.BlockSpec((1, _N, _D), lambda b: (b, 0, 0)),
            pl.BlockSpec((1, 4, _N, 32), lambda b: (b, 0, 0, 0)),
            pl.BlockSpec((1, 4, _N, 32), lambda b: (b, 0, 0, 0)),
        ],
        out_shape=[
            jax.ShapeDtypeStruct((_B, _N, _D), jnp.float32),
            jax.ShapeDtypeStruct((_B, 4, _N, 32), jnp.int32),
            jax.ShapeDtypeStruct((_B, 4, _N, 32), jnp.float32),
        ],
        compiler_params=pltpu.CompilerParams(
            dimension_semantics=("parallel",)),
    )(scal, qe, ctrl8, wq_t, bq, woff_t, boff, wattn_t, battn, m64)


_NW = 32
_PER_W = _S // _NW
_CH = 256


_NBUF = 3


def _sc_gather_body(table_ref, idx_ref, out_ref, idx_v, rows_v,
                    semg0, semg1, semg2, semw0, semw1, semw2):
    wid = lax.axis_index("s") * 2 + lax.axis_index("c")
    base = pl.multiple_of(wid * _PER_W, _CH)
    pltpu.sync_copy(idx_ref.at[pl.ds(base, _PER_W)], idx_v)
    semg = (semg0, semg1, semg2)
    semw = (semw0, semw1, semw2)
    nch = _PER_W // _CH
    pend_g = [None] * _NBUF
    pend_w = [None] * _NBUF
    offs = [None] * _NBUF
    for j in range(nch):
        cur = j % _NBUF
        off = pl.multiple_of(base + j * _CH, _CH)
        if pend_w[cur] is not None:
            pend_w[cur].wait()
            pend_w[cur] = None
        pend_g[cur] = pltpu.async_copy(
            table_ref.at[idx_v.at[pl.ds(j * _CH, _CH)]],
            rows_v.at[cur], semg[cur])
        offs[cur] = off
        if j >= _NBUF - 1:
            d = (j - (_NBUF - 1)) % _NBUF
            pend_g[d].wait()
            pend_g[d] = None
            pend_w[d] = pltpu.async_copy(
                rows_v.at[d], out_ref.at[pl.ds(offs[d], _CH)], semw[d])
    for j in range(nch - (_NBUF - 1), nch):
        d = j % _NBUF
        pend_g[d].wait()
        pend_g[d] = None
        pend_w[d] = pltpu.async_copy(
            rows_v.at[d], out_ref.at[pl.ds(offs[d], _CH)], semw[d])
    for bufi in range(_NBUF):
        if pend_w[bufi] is not None:
            pend_w[bufi].wait()


_sc_gather = functools.partial(
    pl.kernel,
    out_type=jax.ShapeDtypeStruct((_S, 128), jnp.float32),
    mesh=plsc.VectorSubcoreMesh(core_axis_name="c", subcore_axis_name="s"),
    scratch_types=[
        pltpu.VMEM((_PER_W,), jnp.int32),
        pltpu.VMEM((_NBUF, _CH, 128), jnp.float32),
        pltpu.SemaphoreType.DMA,
        pltpu.SemaphoreType.DMA,
        pltpu.SemaphoreType.DMA,
        pltpu.SemaphoreType.DMA,
        pltpu.SemaphoreType.DMA,
        pltpu.SemaphoreType.DMA,
    ],
)(_sc_gather_body)


_NB = 256


def _combine_body(g_ref, w_ref, e2_ref, q_ref, selwm_ref, bm_ref, wo_ref,
                  bo_ref, o_ref):
    u = g_ref[0, 0] * jnp.dot(w_ref[0, 0], e2_ref[...],
                              preferred_element_type=jnp.float32)
    for c in range(1, 4):
        u = u + g_ref[0, c] * jnp.dot(w_ref[0, c], e2_ref[...],
                                      preferred_element_type=jnp.float32)
    y = (jnp.dot(u, selwm_ref[...], preferred_element_type=jnp.float32)
         + bm_ref[...] + q_ref[0])
    o_ref[0] = (jnp.dot(y, wo_ref[...], preferred_element_type=jnp.float32)
                + bo_ref[...])


def _combine(g4, wts, e2, q, selwm, bm, wo_t, bo):
    full = lambda shape: pl.BlockSpec(shape, lambda b, i: (0,) * len(shape))
    return pl.pallas_call(
        _combine_body,
        grid=(_B, _N // _NB),
        in_specs=[
            pl.BlockSpec((1, 4, _NB, 4096), lambda b, i: (b, 0, i, 0)),
            pl.BlockSpec((1, 4, _NB, 32), lambda b, i: (b, 0, i, 0)),
            full((32, 4096)),
            pl.BlockSpec((1, _NB, _D), lambda b, i: (b, i, 0)),
            full((4096, _D)), full((1, _D)),
            full((_D, _D)), full((1, _D)),
        ],
        out_specs=pl.BlockSpec((1, _NB, _D), lambda b, i: (b, i, 0)),
        out_shape=jax.ShapeDtypeStruct((_B, _N, _D), jnp.float32),
        compiler_params=pltpu.CompilerParams(
            dimension_semantics=("parallel", "parallel")),
    )(g4, wts, e2, q, selwm, bm, wo_t, bo)


def kernel(query_embed, ctrl_points, bev_features, spatial_shapes, pc_range,
           W_query, b_query, W_value, b_value, W_off, b_off,
           W_attn, b_attn, W_msda_out, b_msda_out, W_out, b_out):
    del spatial_shapes
    m64 = jnp.asarray(_M64_np)
    e2 = jnp.asarray(_E2_np)
    operm = jnp.asarray(_OPERM_np)
    aperm = jnp.asarray(_APERM_np)
    selwm = (jnp.zeros((4096, _D), jnp.float32)
             .at[_SELROWS_np].set(W_msda_out.T[_SELSRCS_np]))

    value_in = bev_features.reshape(_B, _D, _HW).transpose(0, 2, 1)
    value = _value_proj(value_in, W_value.T, b_value[None])
    table = value.reshape(_B * _HW * 2, 128)

    scal = jnp.stack([pc_range[0], 1.0 / (pc_range[3] - pc_range[0]),
                      pc_range[1], 1.0 / (pc_range[4] - pc_range[1])])
    ctrl8 = ctrl_points.reshape(_B, _N, 8)
    q, idx, wts = _query_side(
        scal, query_embed, ctrl8,
        W_query.T, b_query[None],
        W_off[operm].T, b_off[operm][None],
        W_attn[aperm].T, b_attn[aperm][None],
        m64)

    g = _sc_gather(table, idx.reshape(_S))
    g4 = g.reshape(_B, 4, _N, 32 * 128)

    return _combine(g4, wts, e2, q, selwm, b_msda_out[None], W_out.T, b_out[None])
